# baseline (speedup 1.0000x reference)
import functools

import jax
import jax.numpy as jnp
from jax import lax
from jax.experimental import pallas as pl
from jax.experimental.pallas import tpu as pltpu
from jax.experimental.pallas import tpu_sc as plsc

N = 10000
NP = 10240
E = 160000
D = 256
HD = D // 2
NC = 2
NS = 16
K = 80
NCHUNK = 128
BLKC = 32
EPTP = NCHUNK * K
EP = EPTP * NS
ROWS = NP // NS

_mesh = plsc.VectorSubcoreMesh(
    core_axis_name="c", subcore_axis_name="s", num_cores=NC, num_subcores=NS
)


@functools.partial(
    pl.kernel,
    out_type=jax.ShapeDtypeStruct((NC, NP, HD), jnp.float32),
    mesh=_mesh,
    scratch_types=[
        pltpu.VMEM((NCHUNK + 1, K), jnp.int32),
        pltpu.VMEM((BLKC, K), jnp.int32),
        pltpu.VMEM((K, HD), jnp.float32),
        pltpu.VMEM((K, HD), jnp.float32),
        pltpu.VMEM_SHARED((NP, HD), jnp.float32),
        pltpu.SemaphoreType.DMA,
        pltpu.SemaphoreType.DMA,
    ],
)
def _sc_segsum(
    fcat_hbm, src_hbm, dst_hbm, out_hbm, src_v, dst_v, bufa, bufb, acc, sema, semb
):
    c = lax.axis_index("c")
    s = lax.axis_index("s")
    base = s * ROWS
    pltpu.sync_copy(
        fcat_hbm.at[pl.ds(c * NP + base, ROWS)], acc.at[pl.ds(base, ROWS)]
    )
    pltpu.sync_copy(src_hbm.at[c, s], src_v)
    plsc.subcore_barrier()

    pltpu.async_copy(fcat_hbm.at[src_v.at[0]], bufa, sema)

    def block(b, carry):
        pltpu.sync_copy(dst_hbm.at[s, pl.ds(b * BLKC, BLKC)], dst_v)

        def pair(jp, carry2):
            j = b * BLKC + jp * 2
            l = jp * 2
            pltpu.make_async_copy(fcat_hbm.at[src_v.at[j]], bufa, sema).wait()
            pltpu.async_copy(fcat_hbm.at[src_v.at[j + 1]], bufb, semb)
            pltpu.sync_copy(bufa, acc.at[dst_v.at[l]], add=True)
            pltpu.make_async_copy(fcat_hbm.at[src_v.at[j + 1]], bufb, semb).wait()
            pltpu.async_copy(fcat_hbm.at[src_v.at[j + 2]], bufa, sema)
            pltpu.sync_copy(bufb, acc.at[dst_v.at[l + 1]], add=True)
            return carry2

        lax.fori_loop(0, BLKC // 2, pair, 0)
        return carry

    lax.fori_loop(0, NCHUNK // BLKC, block, 0)
    pltpu.make_async_copy(fcat_hbm.at[src_v.at[0]], bufa, sema).wait()
    plsc.subcore_barrier()
    pltpu.sync_copy(
        acc.at[pl.ds(base, ROWS)], out_hbm.at[c, pl.ds(base, ROWS)]
    )


BN = 1000


def _tc_mlp(h2_ref, wc_ref, bc_ref, wm_ref, bm_ref, out_ref):
    h = jnp.concatenate([h2_ref[0], h2_ref[1]], axis=-1)
    coff = jax.nn.sigmoid(
        jnp.dot(h, wc_ref[...], preferred_element_type=jnp.float32) + bc_ref[...]
    )
    msg = jnp.tanh(
        jnp.dot(h, wm_ref[...], preferred_element_type=jnp.float32) + bm_ref[...]
    )
    out_ref[...] = coff * msg


def kernel(feat, edge_index, W_coff, b_coff, W_msg, b_msg):
    src = edge_index[0]
    dst = edge_index[1]
    fpad = jnp.pad(feat, ((0, NP - N), (0, 0)))
    fcat = jnp.concatenate([fpad[:, :HD], fpad[:, HD:]], axis=0)
    pi = jnp.arange(EP - E, dtype=jnp.int32)
    srcp = jnp.concatenate([src, N + pi % (NP - N)])
    dstp = jnp.concatenate([dst, pi % N])
    src_t = srcp.reshape(NS, NCHUNK, K)
    src_t = jnp.concatenate([src_t, jnp.full((NS, 1, K), N, jnp.int32)], axis=1)
    src2 = jnp.stack([src_t, src_t + NP])
    dst_t = dstp.reshape(NS, NCHUNK, K)

    h2 = _sc_segsum(fcat, src2, dst_t)

    out = pl.pallas_call(
        _tc_mlp,
        grid=(N // BN,),
        in_specs=[
            pl.BlockSpec((NC, BN, HD), lambda i: (0, i, 0)),
            pl.BlockSpec((D, D), lambda i: (0, 0)),
            pl.BlockSpec((1, D), lambda i: (0, 0)),
            pl.BlockSpec((D, D), lambda i: (0, 0)),
            pl.BlockSpec((1, D), lambda i: (0, 0)),
        ],
        out_specs=pl.BlockSpec((BN, D), lambda i: (i, 0)),
        out_shape=jax.ShapeDtypeStruct((N, D), jnp.float32),
    )(h2, W_coff, b_coff.reshape(1, D), W_msg, b_msg.reshape(1, D))
    return out


# speedup vs baseline: 1.0965x; 1.0965x over previous
import functools

import jax
import jax.numpy as jnp
from jax import lax
from jax.experimental import pallas as pl
from jax.experimental.pallas import tpu as pltpu
from jax.experimental.pallas import tpu_sc as plsc

N = 10000
NP = 10240
E = 160000
D = 256
HD = D // 2
NC = 2
NS = 16
K = 128
NCHUNK = 80
EPTP = NCHUNK * K
EP = EPTP * NS
ROWS = NP // NS

_mesh = plsc.VectorSubcoreMesh(
    core_axis_name="c", subcore_axis_name="s", num_cores=NC, num_subcores=NS
)


@functools.partial(
    pl.kernel,
    out_type=jax.ShapeDtypeStruct((NC, NP, HD), jnp.float32),
    mesh=_mesh,
    scratch_types=[
        pltpu.VMEM((NCHUNK, K), jnp.int32),
        pltpu.VMEM((NCHUNK, K), jnp.int32),
        pltpu.VMEM((K, HD), jnp.float32),
        pltpu.VMEM_SHARED((NP, HD), jnp.float32),
        pltpu.SemaphoreType.DMA,
    ],
)
def _sc_segsum(fcat_hbm, src_hbm, dst_hbm, out_hbm, src_v, dst_v, buf, acc, sem):
    c = lax.axis_index("c")
    s = lax.axis_index("s")
    base = s * ROWS
    pltpu.sync_copy(
        fcat_hbm.at[pl.ds(c * NP + base, ROWS)], acc.at[pl.ds(base, ROWS)]
    )
    pltpu.sync_copy(src_hbm.at[c, s], src_v)
    pltpu.sync_copy(dst_hbm.at[s], dst_v)
    plsc.subcore_barrier()

    def chunk(j, carry):
        pltpu.async_copy(fcat_hbm.at[src_v.at[j]], buf, sem).wait()
        pltpu.sync_copy(buf, acc.at[dst_v.at[j]], add=True)
        return carry

    lax.fori_loop(0, NCHUNK, chunk, 0)
    plsc.subcore_barrier()
    pltpu.sync_copy(
        acc.at[pl.ds(base, ROWS)], out_hbm.at[c, pl.ds(base, ROWS)]
    )


BN = 1000


def _tc_mlp(h2_ref, wc_ref, bc_ref, wm_ref, bm_ref, out_ref):
    h = jnp.concatenate([h2_ref[0], h2_ref[1]], axis=-1)
    coff = jax.nn.sigmoid(
        jnp.dot(h, wc_ref[...], preferred_element_type=jnp.float32) + bc_ref[...]
    )
    msg = jnp.tanh(
        jnp.dot(h, wm_ref[...], preferred_element_type=jnp.float32) + bm_ref[...]
    )
    out_ref[...] = coff * msg


def kernel(feat, edge_index, W_coff, b_coff, W_msg, b_msg):
    src = edge_index[0]
    dst = edge_index[1]
    fpad = jnp.pad(feat, ((0, NP - N), (0, 0)))
    fcat = jnp.concatenate([fpad[:, :HD], fpad[:, HD:]], axis=0)
    pi = jnp.arange(EP - E, dtype=jnp.int32)
    srcp = jnp.concatenate([src, N + pi % (NP - N)])
    dstp = jnp.concatenate([dst, pi % N])
    src_t = srcp.reshape(NS, NCHUNK, K)
    src2 = jnp.stack([src_t, src_t + NP])
    dst_t = dstp.reshape(NS, NCHUNK, K)

    h2 = _sc_segsum(fcat, src2, dst_t)

    out = pl.pallas_call(
        _tc_mlp,
        grid=(N // BN,),
        in_specs=[
            pl.BlockSpec((NC, BN, HD), lambda i: (0, i, 0)),
            pl.BlockSpec((D, D), lambda i: (0, 0)),
            pl.BlockSpec((1, D), lambda i: (0, 0)),
            pl.BlockSpec((D, D), lambda i: (0, 0)),
            pl.BlockSpec((1, D), lambda i: (0, 0)),
        ],
        out_specs=pl.BlockSpec((BN, D), lambda i: (i, 0)),
        out_shape=jax.ShapeDtypeStruct((N, D), jnp.float32),
    )(h2, W_coff, b_coff.reshape(1, D), W_msg, b_msg.reshape(1, D))
    return out


# speedup vs baseline: 1.1150x; 1.0169x over previous
import functools

import jax
import jax.numpy as jnp
from jax import lax
from jax.experimental import pallas as pl
from jax.experimental.pallas import tpu as pltpu
from jax.experimental.pallas import tpu_sc as plsc

N = 10000
NP = 10240
E = 160000
D = 256
HD = D // 2
NC = 2
NS = 16
K = 128
NCHUNK = 80
EPTP = NCHUNK * K
EP = EPTP * NS
ROWS = NP // NS

_mesh = plsc.VectorSubcoreMesh(
    core_axis_name="c", subcore_axis_name="s", num_cores=NC, num_subcores=NS
)


@functools.partial(
    pl.kernel,
    out_type=jax.ShapeDtypeStruct((NC, NP, HD), jnp.float32),
    mesh=_mesh,
    scratch_types=[
        pltpu.VMEM((NCHUNK, K), jnp.int32),
        pltpu.VMEM((NCHUNK, K), jnp.int32),
        pltpu.VMEM((K, HD), jnp.float32),
        pltpu.VMEM_SHARED((NP, HD), jnp.float32),
        pltpu.SemaphoreType.DMA,
    ],
)
def _sc_segsum(feat_hbm, fr2_hbm, src_hbm, dst_hbm, out_hbm, src_v, dst_v, buf, acc, sem):
    c = lax.axis_index("c")
    s = lax.axis_index("s")
    base = s * ROWS
    @pl.when(s < NS - 1)
    def _():
        pltpu.sync_copy(
            feat_hbm.at[pl.ds(base, ROWS), pl.ds(c * HD, HD)],
            acc.at[pl.ds(base, ROWS)],
        )

    @pl.when(s == NS - 1)
    def _():
        last = (NS - 1) * ROWS
        pltpu.sync_copy(
            feat_hbm.at[pl.ds(last, N - last), pl.ds(c * HD, HD)],
            acc.at[pl.ds(last, N - last)],
        )

    pltpu.sync_copy(src_hbm.at[c, s], src_v)
    pltpu.sync_copy(dst_hbm.at[s], dst_v)
    plsc.subcore_barrier()

    def chunk(j, carry):
        pltpu.async_copy(fr2_hbm.at[src_v.at[j]], buf, sem).wait()
        pltpu.sync_copy(buf, acc.at[dst_v.at[j]], add=True)
        return carry

    lax.fori_loop(0, NCHUNK, chunk, 0)
    plsc.subcore_barrier()
    pltpu.sync_copy(
        acc.at[pl.ds(base, ROWS)], out_hbm.at[c, pl.ds(base, ROWS)]
    )


BN = 1000


def _tc_mlp(h2_ref, wc_ref, bc_ref, wm_ref, bm_ref, out_ref):
    h = jnp.concatenate([h2_ref[0], h2_ref[1]], axis=-1)
    coff = jax.nn.sigmoid(
        jnp.dot(h, wc_ref[...], preferred_element_type=jnp.float32) + bc_ref[...]
    )
    msg = jnp.tanh(
        jnp.dot(h, wm_ref[...], preferred_element_type=jnp.float32) + bm_ref[...]
    )
    out_ref[...] = coff * msg


def kernel(feat, edge_index, W_coff, b_coff, W_msg, b_msg):
    src = edge_index[0]
    dst = edge_index[1]
    fr2 = feat.reshape(2 * N, HD)
    pi = jnp.arange(EP - E, dtype=jnp.int32)
    srcp = jnp.concatenate([src, pi % N])
    dstp = jnp.concatenate([dst, N + pi % (NP - N)])
    src_t = 2 * srcp.reshape(NS, NCHUNK, K)
    src2 = jnp.stack([src_t, src_t + 1])
    dst_t = dstp.reshape(NS, NCHUNK, K)

    h2 = _sc_segsum(feat, fr2, src2, dst_t)

    out = pl.pallas_call(
        _tc_mlp,
        grid=(N // BN,),
        in_specs=[
            pl.BlockSpec((NC, BN, HD), lambda i: (0, i, 0)),
            pl.BlockSpec((D, D), lambda i: (0, 0)),
            pl.BlockSpec((1, D), lambda i: (0, 0)),
            pl.BlockSpec((D, D), lambda i: (0, 0)),
            pl.BlockSpec((1, D), lambda i: (0, 0)),
        ],
        out_specs=pl.BlockSpec((BN, D), lambda i: (i, 0)),
        out_shape=jax.ShapeDtypeStruct((N, D), jnp.float32),
    )(h2, W_coff, b_coff.reshape(1, D), W_msg, b_msg.reshape(1, D))
    return out


# speedup vs baseline: 1.1155x; 1.0005x over previous
import functools

import jax
import jax.numpy as jnp
from jax import lax
from jax.experimental import pallas as pl
from jax.experimental.pallas import tpu as pltpu
from jax.experimental.pallas import tpu_sc as plsc

N = 10000
NP = 10240
E = 160000
D = 256
HD = D // 2
NC = 2
NS = 16
K = 128
NCHUNK = 80
EPTP = NCHUNK * K
EP = EPTP * NS
ROWS = NP // NS

_mesh = plsc.VectorSubcoreMesh(
    core_axis_name="c", subcore_axis_name="s", num_cores=NC, num_subcores=NS
)


@functools.partial(
    pl.kernel,
    out_type=jax.ShapeDtypeStruct((NC, NP, HD), jnp.float32),
    mesh=_mesh,
    scratch_types=[
        pltpu.VMEM((NCHUNK, K), jnp.int32),
        pltpu.VMEM((NCHUNK, K), jnp.int32),
        pltpu.VMEM((K, HD), jnp.float32),
        pltpu.VMEM_SHARED((NP, HD), jnp.float32),
        pltpu.SemaphoreType.DMA,
    ],
)
def _sc_segsum(feat_hbm, fr2_hbm, src_hbm, dst_hbm, out_hbm, src_v, dst_v, buf, acc, sem):
    c = lax.axis_index("c")
    s = lax.axis_index("s")
    base = s * ROWS
    @pl.when(s < NS - 1)
    def _():
        pltpu.sync_copy(
            feat_hbm.at[pl.ds(base, ROWS), pl.ds(c * HD, HD)],
            acc.at[pl.ds(base, ROWS)],
        )

    @pl.when(s == NS - 1)
    def _():
        last = (NS - 1) * ROWS
        pltpu.sync_copy(
            feat_hbm.at[pl.ds(last, N - last), pl.ds(c * HD, HD)],
            acc.at[pl.ds(last, N - last)],
        )

    pltpu.sync_copy(src_hbm.at[c, s], src_v)
    pltpu.sync_copy(dst_hbm.at[s], dst_v)
    plsc.subcore_barrier()

    @plsc.parallel_loop(0, NCHUNK, 1, unroll=2)
    def _(j):
        pltpu.async_copy(fr2_hbm.at[src_v.at[j]], buf, sem).wait()
        pltpu.sync_copy(buf, acc.at[dst_v.at[j]], add=True)
    plsc.subcore_barrier()
    pltpu.sync_copy(
        acc.at[pl.ds(base, ROWS)], out_hbm.at[c, pl.ds(base, ROWS)]
    )


BN = 1000


def _tc_mlp(h2_ref, wc_ref, bc_ref, wm_ref, bm_ref, out_ref):
    h = jnp.concatenate([h2_ref[0], h2_ref[1]], axis=-1)
    coff = jax.nn.sigmoid(
        jnp.dot(h, wc_ref[...], preferred_element_type=jnp.float32) + bc_ref[...]
    )
    msg = jnp.tanh(
        jnp.dot(h, wm_ref[...], preferred_element_type=jnp.float32) + bm_ref[...]
    )
    out_ref[...] = coff * msg


def kernel(feat, edge_index, W_coff, b_coff, W_msg, b_msg):
    src = edge_index[0]
    dst = edge_index[1]
    fr2 = feat.reshape(2 * N, HD)
    pi = jnp.arange(EP - E, dtype=jnp.int32)
    srcp = jnp.concatenate([src, pi % N])
    dstp = jnp.concatenate([dst, N + pi % (NP - N)])
    src_t = 2 * srcp.reshape(NS, NCHUNK, K)
    src2 = jnp.stack([src_t, src_t + 1])
    dst_t = dstp.reshape(NS, NCHUNK, K)

    h2 = _sc_segsum(feat, fr2, src2, dst_t)

    out = pl.pallas_call(
        _tc_mlp,
        grid=(N // BN,),
        in_specs=[
            pl.BlockSpec((NC, BN, HD), lambda i: (0, i, 0)),
            pl.BlockSpec((D, D), lambda i: (0, 0)),
            pl.BlockSpec((1, D), lambda i: (0, 0)),
            pl.BlockSpec((D, D), lambda i: (0, 0)),
            pl.BlockSpec((1, D), lambda i: (0, 0)),
        ],
        out_specs=pl.BlockSpec((BN, D), lambda i: (i, 0)),
        out_shape=jax.ShapeDtypeStruct((N, D), jnp.float32),
    )(h2, W_coff, b_coff.reshape(1, D), W_msg, b_msg.reshape(1, D))
    return out
